# baseline (speedup 1.0000x reference)
import dataclasses

import jax
import jax.numpy as jnp
from jax import lax
from jax.experimental import pallas as pl
from jax.experimental.pallas import tpu as pltpu
from jax.experimental.pallas import tpu_sc as plsc

N = 10000
E = 160000
D = 256
L = 3
ALPHA = 1.0
BETA = 1.0
SCALING = 1.0

H = D // 2
NSUB = 16
CH = 88
NCHUNK = 120
EPTP = NCHUNK * CH
SBC = 8
NSB = NCHUNK // SBC
WCH = 200
NWC = N // WCH
ZCH = 80
NZC = N // ZCH

assert ALPHA == BETA
_a, _b = ALPHA, BETA
K1 = (_a + _b + 2.0) / 2.0
_c0_2 = 2 * 2 * (2 + _a + _b) * (2 * 2 + _a + _b - 2)
_c2_2 = (2 * 2 + _a + _b - 1) * (2 * 2 + _a + _b) * (2 * 2 + _a + _b - 2)
_c3_2 = 2 * (2 + _a - 1) * (2 + _b - 1) * (2 * 2 + _a + _b)
P2 = _c2_2 * K1 / _c0_2
Q2 = -_c3_2 / _c0_2
_c0_3 = 2 * 3 * (3 + _a + _b) * (2 * 3 + _a + _b - 2)
_c2_3 = (2 * 3 + _a + _b - 1) * (2 * 3 + _a + _b) * (2 * 3 + _a + _b - 2)
_c3_3 = 2 * (3 + _a - 1) * (3 + _b - 1) * (2 * 3 + _a + _b)
P3 = _c2_3 * P2 / _c0_3
R3 = _c2_3 * Q2 / _c0_3
Q3 = -_c3_3 * K1 / _c0_3


def _one_pass(tbl_hbm, out_hbm, src_hbm, dst_hbm, w_hbm, zero_hbm,
              idx_v, dst_v, w_v, gbufs, acc, gsem, ssem, c, s):
    for k in range(-(-NZC // NSUB)):
        zchunk = k * NSUB + s

        @pl.when(zchunk < NZC)
        def _():
            pltpu.sync_copy(zero_hbm, acc.at[pl.ds(zchunk * ZCH, ZCH)])
    plsc.subcore_barrier()

    @pl.loop(0, NSB)
    def _sb(sb):
        pltpu.sync_copy(src_hbm.at[c, s, sb], idx_v)
        pltpu.sync_copy(dst_hbm.at[s, sb], dst_v)
        pltpu.sync_copy(w_hbm.at[s, sb], w_v)

        for b in range(2):
            pltpu.async_copy(tbl_hbm.at[idx_v.at[b]], gbufs[b], gsem.at[b])

        @pl.loop(0, SBC, step=4)
        def _trip(ci):
            for b in range(4):
                gbuf = gbufs[b]
                cur = ci + b
                nb = (b + 2) % 4

                @pl.when(cur >= 2)
                def _():
                    pltpu.make_async_copy(gbufs[nb],
                                          acc.at[dst_v.at[cur - 2]],
                                          ssem.at[nb]).wait()

                @pl.when(cur + 2 < SBC)
                def _():
                    pltpu.async_copy(tbl_hbm.at[idx_v.at[cur + 2]],
                                     gbufs[nb], gsem.at[nb])

                pltpu.make_async_copy(tbl_hbm.at[idx_v.at[cur]],
                                      gbuf, gsem.at[b]).wait()

                ci16 = jnp.full((16,), cur, jnp.int32)

                @pl.loop(0, CH)
                def _row(k):
                    wv = plsc.load_gather(
                        w_v, [ci16, jnp.full((16,), k, jnp.int32)])
                    for j in range(H // 16):
                        sl = pl.ds(16 * j, 16)
                        gbuf[k, sl] = gbuf[k, sl] * wv

                pltpu.async_copy(gbuf, acc.at[dst_v.at[cur]],
                                 ssem.at[b], add=True)

        for last in (SBC - 2, SBC - 1):
            pltpu.make_async_copy(gbufs[last % 4], acc.at[dst_v.at[last]],
                                  ssem.at[last % 4]).wait()

    plsc.subcore_barrier()

    for k in range(-(-NWC // NSUB)):
        chunk = k * NSUB + s

        @pl.when(chunk < NWC)
        def _():
            pltpu.sync_copy(acc.at[pl.ds(chunk * WCH, WCH)],
                            out_hbm.at[pl.ds(c * N + chunk * WCH, WCH)])
    plsc.subcore_barrier()


def _spmm_body(src_hbm, dst_hbm, w_hbm, tbl_hbm, zero_hbm, s1_hbm, s2_hbm,
               idx_v, dst_v, w_v, g0_v, g1_v, g2_v, g3_v, acc, gsem, ssem):
    gbufs = (g0_v, g1_v, g2_v, g3_v)
    c = lax.axis_index("c")
    s = lax.axis_index("s")
    common = (src_hbm, dst_hbm, w_hbm, zero_hbm,
              idx_v, dst_v, w_v, gbufs, acc, gsem, ssem, c, s)
    _one_pass(tbl_hbm, s1_hbm, *common)
    _one_pass(s1_hbm, s2_hbm, *common)


_SC_PARAMS = pltpu.CompilerParams()
if "needs_layout_passes" in pltpu.CompilerParams.__dataclass_fields__:
    _SC_PARAMS = dataclasses.replace(_SC_PARAMS, needs_layout_passes=False)


def _spmm2(tbl2, srcadj, dst3, w3, zeros):
    kfn = pl.kernel(
        _spmm_body,
        out_type=[jax.ShapeDtypeStruct((2 * N, H), jnp.float32),
                  jax.ShapeDtypeStruct((2 * N, H), jnp.float32)],
        mesh=plsc.VectorSubcoreMesh(core_axis_name="c", subcore_axis_name="s"),
        scratch_types=[
            pltpu.VMEM((SBC, CH), jnp.int32),
            pltpu.VMEM((SBC, CH), jnp.int32),
            pltpu.VMEM((SBC, CH), jnp.float32),
            pltpu.VMEM((CH, H), jnp.float32),
            pltpu.VMEM((CH, H), jnp.float32),
            pltpu.VMEM((CH, H), jnp.float32),
            pltpu.VMEM((CH, H), jnp.float32),
            pltpu.VMEM_SHARED((N, H), jnp.float32),
            pltpu.SemaphoreType.DMA((4,)),
            pltpu.SemaphoreType.DMA((4,)),
        ],
        compiler_params=_SC_PARAMS,
    )
    return kfn(srcadj, dst3, w3, tbl2, zeros)


def _combine_body(g_ref, xlo, xhi, s1lo, s1hi, s2lo, s2hi, o_ref):
    t = jnp.tanh(g_ref[...]) * SCALING
    c0 = t[0:1, :]
    c1 = c0 * t[1:2, :]
    c2 = c1 * t[2:3, :]
    c3 = c2 * t[3:4, :]
    A = c0 + K1 * c1 + Q2 * c2 + Q3 * c3
    B = P2 * c2 + R3 * c3
    C = P3 * c3
    o_ref[:, :H] = A[:, :H] * xlo[...] + B[:, :H] * s1lo[...] + C[:, :H] * s2lo[...]
    o_ref[:, H:] = A[:, H:] * xhi[...] + B[:, H:] * s1hi[...] + C[:, H:] * s2hi[...]


def _combine(gammas, xh2, s1, s2):
    R = 1000
    nblk = N // R

    def lo(i):
        return (i, 0)

    def hi(i):
        return (i + nblk, 0)

    half = lambda imap: pl.BlockSpec((R, H), imap)
    return pl.pallas_call(
        _combine_body,
        grid=(nblk,),
        in_specs=[
            pl.BlockSpec((L + 1, D), lambda i: (0, 0)),
            half(lo), half(hi), half(lo), half(hi), half(lo), half(hi),
        ],
        out_specs=pl.BlockSpec((R, D), lambda i: (i, 0)),
        out_shape=jax.ShapeDtypeStruct((N, D), jnp.float32),
    )(gammas, xh2, xh2, s1, s1, s2, s2)


def kernel(x, edge_index, edge_weight, gammas):
    src = edge_index[0].astype(jnp.int32)
    dst = edge_index[1].astype(jnp.int32)
    xh2 = jnp.concatenate([x[:, :H], x[:, H:]], axis=0)
    npad = NSUB * EPTP - E
    fill = (jnp.arange(npad, dtype=jnp.int32) * 37) % N
    src_p = jnp.concatenate([src, fill])
    dst_p = jnp.concatenate([dst, fill])
    w_p = jnp.concatenate([edge_weight, jnp.zeros((npad,), jnp.float32)])
    src4 = src_p.reshape(NSUB, NSB, SBC, CH)
    srcadj = jnp.stack([src4, src4 + N], axis=0)
    dst4 = dst_p.reshape(NSUB, NSB, SBC, CH)
    w4 = w_p.reshape(NSUB, NSB, SBC, CH)
    zeros = jnp.zeros((ZCH, H), jnp.float32)
    s1, s2 = _spmm2(xh2, srcadj, dst4, w4, zeros)
    return _combine(gammas, xh2, s1, s2)


# speedup vs baseline: 1.1666x; 1.1666x over previous
import dataclasses

import jax
import jax.numpy as jnp
from jax import lax
from jax.experimental import pallas as pl
from jax.experimental.pallas import tpu as pltpu
from jax.experimental.pallas import tpu_sc as plsc

N = 10000
E = 160000
D = 256
L = 3
ALPHA = 1.0
BETA = 1.0
SCALING = 1.0

H = D // 2
NSUB = 16
CH = 112
NCHUNK = 90
EPTP = NCHUNK * CH
SBC = 15
NSB = NCHUNK // SBC
WCH = 200
NWC = N // WCH
ZCH = 80
NZC = N // ZCH

assert ALPHA == BETA
_a, _b = ALPHA, BETA
K1 = (_a + _b + 2.0) / 2.0
_c0_2 = 2 * 2 * (2 + _a + _b) * (2 * 2 + _a + _b - 2)
_c2_2 = (2 * 2 + _a + _b - 1) * (2 * 2 + _a + _b) * (2 * 2 + _a + _b - 2)
_c3_2 = 2 * (2 + _a - 1) * (2 + _b - 1) * (2 * 2 + _a + _b)
P2 = _c2_2 * K1 / _c0_2
Q2 = -_c3_2 / _c0_2
_c0_3 = 2 * 3 * (3 + _a + _b) * (2 * 3 + _a + _b - 2)
_c2_3 = (2 * 3 + _a + _b - 1) * (2 * 3 + _a + _b) * (2 * 3 + _a + _b - 2)
_c3_3 = 2 * (3 + _a - 1) * (3 + _b - 1) * (2 * 3 + _a + _b)
P3 = _c2_3 * P2 / _c0_3
R3 = _c2_3 * Q2 / _c0_3
Q3 = -_c3_3 * K1 / _c0_3


def _one_pass(tbl_hbm, out_hbm, src_hbm, dst_hbm, w_hbm, zero_hbm,
              idx_v, dst_v, w_v, gbufs, acc, gsem, ssem, c, s):
    for k in range(-(-NZC // NSUB)):
        zchunk = k * NSUB + s

        @pl.when(zchunk < NZC)
        def _():
            pltpu.sync_copy(zero_hbm, acc.at[pl.ds(zchunk * ZCH, ZCH)])
    plsc.subcore_barrier()

    @pl.loop(0, NSB)
    def _sb(sb):
        pltpu.sync_copy(src_hbm.at[c, s, sb], idx_v)
        pltpu.sync_copy(dst_hbm.at[s, sb], dst_v)
        pltpu.sync_copy(w_hbm.at[s, sb], w_v)

        pltpu.async_copy(tbl_hbm.at[idx_v.at[0]], gbufs[0], gsem.at[0])

        @pl.loop(0, SBC, step=3)
        def _trip(ci):
            for b in range(3):
                gbuf = gbufs[b]
                cur = ci + b
                nb = (b + 1) % 3

                @pl.when(cur >= 2)
                def _():
                    pltpu.make_async_copy(gbufs[nb],
                                          acc.at[dst_v.at[cur - 2]],
                                          ssem.at[nb]).wait()

                @pl.when(cur + 1 < SBC)
                def _():
                    pltpu.async_copy(tbl_hbm.at[idx_v.at[cur + 1]],
                                     gbufs[nb], gsem.at[nb])

                pltpu.make_async_copy(tbl_hbm.at[idx_v.at[cur]],
                                      gbuf, gsem.at[b]).wait()

                ci16 = jnp.full((16,), cur, jnp.int32)

                @pl.loop(0, CH)
                def _row(k):
                    wv = plsc.load_gather(
                        w_v, [ci16, jnp.full((16,), k, jnp.int32)])
                    for j in range(H // 16):
                        sl = pl.ds(16 * j, 16)
                        gbuf[k, sl] = gbuf[k, sl] * wv

                pltpu.async_copy(gbuf, acc.at[dst_v.at[cur]],
                                 ssem.at[b], add=True)

        for last in (SBC - 2, SBC - 1):
            pltpu.make_async_copy(gbufs[last % 3], acc.at[dst_v.at[last]],
                                  ssem.at[last % 3]).wait()

    plsc.subcore_barrier()

    for k in range(-(-NWC // NSUB)):
        chunk = k * NSUB + s

        @pl.when(chunk < NWC)
        def _():
            pltpu.sync_copy(acc.at[pl.ds(chunk * WCH, WCH)],
                            out_hbm.at[pl.ds(c * N + chunk * WCH, WCH)])
    plsc.subcore_barrier()


def _spmm_body(src_hbm, dst_hbm, w_hbm, tbl_hbm, zero_hbm, s1_hbm, s2_hbm,
               idx_v, dst_v, w_v, g0_v, g1_v, g2_v, acc, gsem, ssem):
    gbufs = (g0_v, g1_v, g2_v)
    c = lax.axis_index("c")
    s = lax.axis_index("s")
    common = (src_hbm, dst_hbm, w_hbm, zero_hbm,
              idx_v, dst_v, w_v, gbufs, acc, gsem, ssem, c, s)
    _one_pass(tbl_hbm, s1_hbm, *common)
    _one_pass(s1_hbm, s2_hbm, *common)


_SC_PARAMS = pltpu.CompilerParams()
if "needs_layout_passes" in pltpu.CompilerParams.__dataclass_fields__:
    _SC_PARAMS = dataclasses.replace(_SC_PARAMS, needs_layout_passes=False)


def _spmm2(tbl2, srcadj, dst3, w3, zeros):
    kfn = pl.kernel(
        _spmm_body,
        out_type=[jax.ShapeDtypeStruct((2 * N, H), jnp.float32),
                  jax.ShapeDtypeStruct((2 * N, H), jnp.float32)],
        mesh=plsc.VectorSubcoreMesh(core_axis_name="c", subcore_axis_name="s"),
        scratch_types=[
            pltpu.VMEM((SBC, CH), jnp.int32),
            pltpu.VMEM((SBC, CH), jnp.int32),
            pltpu.VMEM((SBC, CH), jnp.float32),
            pltpu.VMEM((CH, H), jnp.float32),
            pltpu.VMEM((CH, H), jnp.float32),
            pltpu.VMEM((CH, H), jnp.float32),
            pltpu.VMEM_SHARED((N, H), jnp.float32),
            pltpu.SemaphoreType.DMA((3,)),
            pltpu.SemaphoreType.DMA((3,)),
        ],
        compiler_params=_SC_PARAMS,
    )
    return kfn(srcadj, dst3, w3, tbl2, zeros)


def _combine_body(g_ref, xlo, xhi, s1lo, s1hi, s2lo, s2hi, o_ref):
    t = jnp.tanh(g_ref[...]) * SCALING
    c0 = t[0:1, :]
    c1 = c0 * t[1:2, :]
    c2 = c1 * t[2:3, :]
    c3 = c2 * t[3:4, :]
    A = c0 + K1 * c1 + Q2 * c2 + Q3 * c3
    B = P2 * c2 + R3 * c3
    C = P3 * c3
    o_ref[:, :H] = A[:, :H] * xlo[...] + B[:, :H] * s1lo[...] + C[:, :H] * s2lo[...]
    o_ref[:, H:] = A[:, H:] * xhi[...] + B[:, H:] * s1hi[...] + C[:, H:] * s2hi[...]


def _combine(gammas, xh2, s1, s2):
    R = 1000
    nblk = N // R

    def lo(i):
        return (i, 0)

    def hi(i):
        return (i + nblk, 0)

    half = lambda imap: pl.BlockSpec((R, H), imap)
    return pl.pallas_call(
        _combine_body,
        grid=(nblk,),
        in_specs=[
            pl.BlockSpec((L + 1, D), lambda i: (0, 0)),
            half(lo), half(hi), half(lo), half(hi), half(lo), half(hi),
        ],
        out_specs=pl.BlockSpec((R, D), lambda i: (i, 0)),
        out_shape=jax.ShapeDtypeStruct((N, D), jnp.float32),
    )(gammas, xh2, xh2, s1, s1, s2, s2)


def kernel(x, edge_index, edge_weight, gammas):
    src = edge_index[0].astype(jnp.int32)
    dst = edge_index[1].astype(jnp.int32)
    xh2 = jnp.concatenate([x[:, :H], x[:, H:]], axis=0)
    npad = NSUB * EPTP - E
    fill = (jnp.arange(npad, dtype=jnp.int32) * 37) % N
    src_p = jnp.concatenate([src, fill])
    dst_p = jnp.concatenate([dst, fill])
    w_p = jnp.concatenate([edge_weight, jnp.zeros((npad,), jnp.float32)])
    src4 = src_p.reshape(NSUB, NSB, SBC, CH)
    srcadj = jnp.stack([src4, src4 + N], axis=0)
    dst4 = dst_p.reshape(NSUB, NSB, SBC, CH)
    w4 = w_p.reshape(NSUB, NSB, SBC, CH)
    zeros = jnp.zeros((ZCH, H), jnp.float32)
    s1, s2 = _spmm2(xh2, srcadj, dst4, w4, zeros)
    return _combine(gammas, xh2, s1, s2)
